# Initial kernel scaffold; baseline (speedup 1.0000x reference)
import jax
import jax.numpy as jnp
from jax.experimental import pallas as pl


def kernel(emb, W_gc0, b_gc0, W_bi0, b_bi0, W_gc1, b_gc1, W_bi1, b_bi1, u, i, j, edge_index):
    raise NotImplementedError("write your pallas kernel here")


# speedup vs baseline: 12.2780x; 12.2780x over previous
import functools

import jax
import jax.numpy as jnp
from jax import lax
from jax.experimental import pallas as pl
from jax.experimental.pallas import tpu as pltpu
from jax.experimental.pallas import tpu_sc as plsc

NU = 25000
NN = 50000
NE = 800000
D = 64
HD = 32
B = 512
NC = 2
NS = 16
NW = NC * NS
L = 16

EPW = 25008
PAD = NW * EPW - NE

K = 80
CPR = NE // K
EPT = NE // NS
NCHUNK = EPT // K
CG = 25
RPT = NN // NS
ZR = 125

R = 2048
G = 25
NNP = R * G

_sc_mesh = plsc.VectorSubcoreMesh(
    core_axis_name="c", subcore_axis_name="s", num_cores=NC, num_subcores=NS
)
_sc_params = pltpu.CompilerParams(
    needs_layout_passes=False, use_tc_tiling_on_sc=False)


@functools.partial(
    pl.kernel,
    out_type=jax.ShapeDtypeStruct((NW, NNP), jnp.float32),
    mesh=_sc_mesh,
    compiler_params=_sc_params,
    scratch_types=[
        pltpu.VMEM((EPW,), jnp.int32),
        pltpu.VMEM((NNP,), jnp.float32),
    ],
)
def _deg_kernel(dst_hbm, out_hbm, dst_v, deg_v):
    c = lax.axis_index("c")
    s = lax.axis_index("s")
    wid = s * NC + c

    zeros = jnp.zeros((L,), jnp.float32)

    def zero_it(g, carry):
        deg_v[pl.ds(g * L, L)] = zeros
        return carry

    lax.fori_loop(0, NNP // L, zero_it, 0)

    pltpu.sync_copy(dst_hbm.at[wid], dst_v)

    ones = jnp.ones((L,), jnp.float32)

    def acc_it(g, carry):
        idx = dst_v[pl.ds(g * L, L)]
        plsc.addupdate_scatter(deg_v, [idx], ones)
        return carry

    lax.fori_loop(0, EPW // L, acc_it, 0)

    pltpu.sync_copy(deg_v, out_hbm.at[wid])


@functools.partial(
    pl.kernel,
    out_type=jax.ShapeDtypeStruct((NC, NN, HD), jnp.float32),
    mesh=_sc_mesh,
    compiler_params=_sc_params,
    scratch_types=[
        pltpu.VMEM((CG, K), jnp.int32),
        pltpu.VMEM((CG, K), jnp.int32),
        pltpu.VMEM((K, HD), jnp.float32),
        pltpu.VMEM((ZR, HD), jnp.float32),
        pltpu.VMEM_SHARED((NN, HD), jnp.float32),
        pltpu.SemaphoreType.DMA,
    ],
)
def _propagate_kernel(src2_hbm, dst2_hbm, table_hbm, out_hbm,
                      src_v, dst_v, rows_v, buf_v, acc_s, sem):
    c = lax.axis_index("c")
    s = lax.axis_index("s")

    zeros = jnp.zeros((L,), jnp.float32)

    def zero_buf(i, carry):
        buf_v[i, pl.ds(0, L)] = zeros
        buf_v[i, pl.ds(L, L)] = zeros
        return carry

    lax.fori_loop(0, ZR, zero_buf, 0)

    def zero_stripe(r, carry):
        pltpu.sync_copy(buf_v, acc_s.at[pl.ds(s * RPT + r * ZR, ZR)])
        return carry

    lax.fori_loop(0, RPT // ZR, zero_stripe, 0)
    plsc.subcore_barrier()

    def group(o, carry):
        g0 = s * NCHUNK + o * CG
        pltpu.sync_copy(src2_hbm.at[c, pl.ds(g0, CG)], src_v)
        pltpu.sync_copy(dst2_hbm.at[pl.ds(g0, CG)], dst_v)

        def step(t, carry2):
            pltpu.async_copy(table_hbm.at[src_v.at[t]], rows_v, sem).wait()
            pltpu.sync_copy(rows_v, acc_s.at[dst_v.at[t]], add=True)
            return carry2

        return lax.fori_loop(0, CG, step, carry)

    lax.fori_loop(0, NCHUNK // CG, group, 0)
    plsc.subcore_barrier()

    def writeout(r, carry):
        r0 = s * RPT + r * ZR
        pltpu.sync_copy(acc_s.at[pl.ds(r0, ZR)], buf_v)
        pltpu.sync_copy(buf_v, out_hbm.at[c, pl.ds(r0, ZR)])
        return carry

    lax.fori_loop(0, RPT // ZR, writeout, 0)


@functools.partial(
    pl.kernel,
    out_type=jax.ShapeDtypeStruct((3, 3, B, D), jnp.float32),
    mesh=_sc_mesh,
    compiler_params=_sc_params,
    scratch_types=[
        pltpu.VMEM((L,), jnp.int32),
        pltpu.VMEM((L, D), jnp.float32),
        pltpu.SemaphoreType.DMA,
    ],
)
def _gather_kernel(t0_hbm, t1_hbm, t2_hbm, uij_hbm, out_hbm,
                   idx_v, rows_v, sem):
    c = lax.axis_index("c")
    s = lax.axis_index("s")
    wid = s * NC + c
    for q in range(3):
        pltpu.sync_copy(uij_hbm.at[q, wid], idx_v)
        for t, tbl in enumerate((t0_hbm, t1_hbm, t2_hbm)):
            pltpu.async_copy(tbl.at[idx_v], rows_v, sem).wait()
            pltpu.sync_copy(rows_v, out_hbm.at[t, q, pl.ds(wid * L, L)])


def _tc0_body(degp_ref, emb_ref, dinv_ref, sstack_ref):
    deg = jnp.sum(degp_ref[...], axis=0)
    dinv = jnp.where(deg > 0, lax.rsqrt(deg), 0.0)[:, None]
    dinv_ref[...] = dinv
    scaled = emb_ref[...] * dinv
    sstack_ref[0] = scaled[:, :HD]
    sstack_ref[1] = scaled[:, HD:]


_tc0 = pl.pallas_call(
    _tc0_body,
    grid=(G,),
    in_specs=[
        pl.BlockSpec((NW, R), lambda g: (0, g)),
        pl.BlockSpec((R, D), lambda g: (g, 0)),
    ],
    out_specs=[
        pl.BlockSpec((R, 1), lambda g: (g, 0)),
        pl.BlockSpec((2, R, HD), lambda g: (0, g, 0)),
    ],
    out_shape=[
        jax.ShapeDtypeStruct((NN, 1), jnp.float32),
        jax.ShapeDtypeStruct((2, NN, HD), jnp.float32),
    ],
)


def _dense_body(last, acc_ref, dinv_ref, ego_ref, wgc_ref, bgc_ref,
                wbi_ref, bbi_ref, *out_refs):
    dinv = dinv_ref[...]
    agg = jnp.concatenate([acc_ref[0], acc_ref[1]], axis=1) * dinv
    ego = ego_ref[...]
    side = agg + ego
    x1 = jnp.dot(side, wgc_ref[...], preferred_element_type=jnp.float32)
    x1 = x1 + bgc_ref[...]
    sum_emb = jnp.where(x1 >= 0, x1, 0.2 * x1)
    x2 = jnp.dot(ego * side, wbi_ref[...], preferred_element_type=jnp.float32)
    x2 = x2 + bbi_ref[...]
    bi_emb = jnp.where(x2 >= 0, x2, 0.2 * x2)
    ego2 = sum_emb + bi_emb
    nrm = ego2 / (jnp.sqrt(jnp.sum(ego2 * ego2, axis=1, keepdims=True)) + 1e-12)
    if last:
        out_refs[0][...] = nrm
    else:
        out_refs[0][...] = ego2
        out_refs[1][...] = nrm
        scaled = ego2 * dinv
        out_refs[2][0] = scaled[:, :HD]
        out_refs[2][1] = scaled[:, HD:]


def _make_dense(last):
    out_specs = [pl.BlockSpec((R, D), lambda g: (g, 0))]
    out_shape = [jax.ShapeDtypeStruct((NN, D), jnp.float32)]
    if not last:
        out_specs = out_specs + [
            pl.BlockSpec((R, D), lambda g: (g, 0)),
            pl.BlockSpec((2, R, HD), lambda g: (0, g, 0)),
        ]
        out_shape = out_shape + [
            jax.ShapeDtypeStruct((NN, D), jnp.float32),
            jax.ShapeDtypeStruct((2, NN, HD), jnp.float32),
        ]
    return pl.pallas_call(
        functools.partial(_dense_body, last),
        grid=(G,),
        in_specs=[
            pl.BlockSpec((2, R, HD), lambda g: (0, g, 0)),
            pl.BlockSpec((R, 1), lambda g: (g, 0)),
            pl.BlockSpec((R, D), lambda g: (g, 0)),
            pl.BlockSpec((D, D), lambda g: (0, 0)),
            pl.BlockSpec((1, D), lambda g: (0, 0)),
            pl.BlockSpec((D, D), lambda g: (0, 0)),
            pl.BlockSpec((1, D), lambda g: (0, 0)),
        ],
        out_specs=out_specs,
        out_shape=out_shape,
    )


_dense0 = _make_dense(last=False)
_dense1 = _make_dense(last=True)


def _dot_body(g_ref, y_ref):
    yui = jnp.zeros((B,), jnp.float32)
    yuj = jnp.zeros((B,), jnp.float32)
    for t in range(3):
        u_rows = g_ref[t, 0]
        yui = yui + jnp.sum(u_rows * g_ref[t, 1], axis=-1)
        yuj = yuj + jnp.sum(u_rows * g_ref[t, 2], axis=-1)
    y_ref[...] = jnp.stack([yui, yuj], axis=1)


_tc_dot = pl.pallas_call(
    _dot_body,
    grid=(1,),
    in_specs=[pl.BlockSpec((3, 3, B, D), lambda g: (0, 0, 0, 0))],
    out_specs=pl.BlockSpec((B, 2), lambda g: (0, 0)),
    out_shape=jax.ShapeDtypeStruct((B, 2), jnp.float32),
)


def kernel(emb, W_gc0, b_gc0, W_bi0, b_bi0, W_gc1, b_gc1, W_bi1, b_bi1,
           u, i, j, edge_index):
    ei = edge_index.astype(jnp.int32)
    src = ei[0]
    dst = ei[1]

    dst_pad = jnp.concatenate(
        [dst, jnp.full((PAD,), NN, jnp.int32)]).reshape(NW, EPW)
    degp = _deg_kernel(dst_pad)

    dinv, s_stack = _tc0(degp, emb)

    src2 = jnp.stack([src, src + NN]).reshape(NC, CPR, K)
    dst2 = dst.reshape(CPR, K)

    acc0 = _propagate_kernel(src2, dst2, s_stack.reshape(NC * NN, HD))
    ego1, n0, s_stack1 = _dense0(
        acc0, dinv, emb, W_gc0, b_gc0.reshape(1, D), W_bi0, b_bi0.reshape(1, D))

    acc1 = _propagate_kernel(src2, dst2, s_stack1.reshape(NC * NN, HD))
    (n1,) = _dense1(
        acc1, dinv, ego1, W_gc1, b_gc1.reshape(1, D), W_bi1, b_bi1.reshape(1, D))

    uij = jnp.stack([
        u.astype(jnp.int32),
        NU + i.astype(jnp.int32),
        NU + j.astype(jnp.int32),
    ]).reshape(3, NW, L)
    g = _gather_kernel(emb, n0, n1, uij)
    return _tc_dot(g)


# speedup vs baseline: 23.9535x; 1.9509x over previous
import functools

import jax
import jax.numpy as jnp
from jax import lax
from jax.experimental import pallas as pl
from jax.experimental.pallas import tpu as pltpu
from jax.experimental.pallas import tpu_sc as plsc

NU = 25000
NN = 50000
NE = 800000
D = 64
HD = 32
B = 512
NC = 2
NS = 16
NW = NC * NS
L = 16

EPW = 25008
PAD = NW * EPW - NE

K = 250
CPR = NE // K
EPT = NE // NS
NCHUNK = EPT // K
CG = 10
RPT = NN // NS
ZR = 125

R = 2048
G = 25
NNP = R * G

_sc_mesh = plsc.VectorSubcoreMesh(
    core_axis_name="c", subcore_axis_name="s", num_cores=NC, num_subcores=NS
)
_sc_params = pltpu.CompilerParams(
    needs_layout_passes=False, use_tc_tiling_on_sc=False)


@functools.partial(
    pl.kernel,
    out_type=jax.ShapeDtypeStruct((NW, NNP), jnp.float32),
    mesh=_sc_mesh,
    compiler_params=_sc_params,
    scratch_types=[
        pltpu.VMEM((EPW,), jnp.int32),
        pltpu.VMEM((NNP,), jnp.float32),
    ],
)
def _deg_kernel(dst_hbm, out_hbm, dst_v, deg_v):
    c = lax.axis_index("c")
    s = lax.axis_index("s")
    wid = s * NC + c

    zeros = jnp.zeros((L,), jnp.float32)

    def zero_it(g, carry):
        deg_v[pl.ds(g * L, L)] = zeros
        return carry

    lax.fori_loop(0, NNP // L, zero_it, 0)

    pltpu.sync_copy(dst_hbm.at[wid], dst_v)

    ones = jnp.ones((L,), jnp.float32)

    def acc_it(g, carry):
        idx = dst_v[pl.ds(g * L, L)]
        plsc.addupdate_scatter(deg_v, [idx], ones)
        return carry

    lax.fori_loop(0, EPW // L, acc_it, 0)

    pltpu.sync_copy(deg_v, out_hbm.at[wid])


@functools.partial(
    pl.kernel,
    out_type=jax.ShapeDtypeStruct((NC, NN, HD), jnp.float32),
    mesh=_sc_mesh,
    compiler_params=_sc_params,
    scratch_types=[
        pltpu.VMEM((2, CG, K), jnp.int32),
        pltpu.VMEM((2, CG, K), jnp.int32),
        pltpu.VMEM((K, HD), jnp.float32),
        pltpu.VMEM((K, HD), jnp.float32),
        pltpu.VMEM_SHARED((NN, HD), jnp.float32),
        pltpu.SemaphoreType.DMA,
        pltpu.SemaphoreType.DMA,
    ],
)
def _propagate_kernel(src2_hbm, dst2_hbm, table_hbm, out_hbm,
                      src_v, dst_v, rows0_v, rows1_v, acc_s, sem0, sem1):
    c = lax.axis_index("c")
    s = lax.axis_index("s")
    rows = (rows0_v, rows1_v)
    sems = (sem0, sem1)

    zeros = jnp.zeros((L,), jnp.float32)

    def zero_buf(i, carry):
        rows0_v[i, pl.ds(0, L)] = zeros
        rows0_v[i, pl.ds(L, L)] = zeros
        return carry

    lax.fori_loop(0, ZR, zero_buf, 0)

    def zero_stripe(r, carry):
        pltpu.sync_copy(rows0_v.at[pl.ds(0, ZR)],
                        acc_s.at[pl.ds(s * RPT + r * ZR, ZR)])
        return carry

    lax.fori_loop(0, RPT // ZR, zero_stripe, 0)
    plsc.subcore_barrier()

    base = s * NCHUNK

    def load_group(o):
        g0 = base + o * CG
        pltpu.sync_copy(src2_hbm.at[c, pl.ds(g0, CG)], src_v.at[o % 2])
        pltpu.sync_copy(dst2_hbm.at[pl.ds(g0, CG)], dst_v.at[o % 2])

    def issue_gather(m, b):
        o = m // CG
        pltpu.async_copy(
            table_hbm.at[src_v.at[o % 2, m % CG]], rows[b], sems[b])

    load_group(0)
    issue_gather(0, 0)
    issue_gather(1, 1)

    def chunk_pair(p, carry):
        def one(m, b):
            o = m // CG
            pltpu.make_async_copy(
                table_hbm.at[src_v.at[o % 2, m % CG]], rows[b],
                sems[b]).wait()
            pltpu.sync_copy(rows[b], acc_s.at[dst_v.at[o % 2, m % CG]],
                            add=True)
            m2 = m + 2

            @pl.when(m2 < NCHUNK)
            def _():
                @pl.when(m2 % CG == 0)
                def _():
                    load_group(m2 // CG)
                issue_gather(m2, b)

        one(2 * p, 0)
        one(2 * p + 1, 1)
        return carry

    lax.fori_loop(0, NCHUNK // 2, chunk_pair, 0)
    plsc.subcore_barrier()

    def writeout(r, carry):
        r0 = s * RPT + r * ZR
        pltpu.sync_copy(acc_s.at[pl.ds(r0, ZR)], rows0_v.at[pl.ds(0, ZR)])
        pltpu.sync_copy(rows0_v.at[pl.ds(0, ZR)], out_hbm.at[c, pl.ds(r0, ZR)])
        return carry

    lax.fori_loop(0, RPT // ZR, writeout, 0)


@functools.partial(
    pl.kernel,
    out_type=jax.ShapeDtypeStruct((3, 3, B, D), jnp.float32),
    mesh=_sc_mesh,
    compiler_params=_sc_params,
    scratch_types=[
        pltpu.VMEM((L,), jnp.int32),
        pltpu.VMEM((L, D), jnp.float32),
        pltpu.SemaphoreType.DMA,
    ],
)
def _gather_kernel(t0_hbm, t1_hbm, t2_hbm, uij_hbm, out_hbm,
                   idx_v, rows_v, sem):
    c = lax.axis_index("c")
    s = lax.axis_index("s")
    wid = s * NC + c
    for q in range(3):
        pltpu.sync_copy(uij_hbm.at[q, wid], idx_v)
        for t, tbl in enumerate((t0_hbm, t1_hbm, t2_hbm)):
            pltpu.async_copy(tbl.at[idx_v], rows_v, sem).wait()
            pltpu.sync_copy(rows_v, out_hbm.at[t, q, pl.ds(wid * L, L)])


def _tc0_body(degp_ref, emb_ref, dinv_ref, sstack_ref):
    deg = jnp.sum(degp_ref[...], axis=0)
    dinv = jnp.where(deg > 0, lax.rsqrt(deg), 0.0)[:, None]
    dinv_ref[...] = dinv
    scaled = emb_ref[...] * dinv
    sstack_ref[0] = scaled[:, :HD]
    sstack_ref[1] = scaled[:, HD:]


_tc0 = pl.pallas_call(
    _tc0_body,
    grid=(G,),
    in_specs=[
        pl.BlockSpec((NW, R), lambda g: (0, g)),
        pl.BlockSpec((R, D), lambda g: (g, 0)),
    ],
    out_specs=[
        pl.BlockSpec((R, 1), lambda g: (g, 0)),
        pl.BlockSpec((2, R, HD), lambda g: (0, g, 0)),
    ],
    out_shape=[
        jax.ShapeDtypeStruct((NN, 1), jnp.float32),
        jax.ShapeDtypeStruct((2, NN, HD), jnp.float32),
    ],
)


def _dense_body(last, acc_ref, dinv_ref, ego_ref, wgc_ref, bgc_ref,
                wbi_ref, bbi_ref, *out_refs):
    dinv = dinv_ref[...]
    agg = jnp.concatenate([acc_ref[0], acc_ref[1]], axis=1) * dinv
    ego = ego_ref[...]
    side = agg + ego
    x1 = jnp.dot(side, wgc_ref[...], preferred_element_type=jnp.float32)
    x1 = x1 + bgc_ref[...]
    sum_emb = jnp.where(x1 >= 0, x1, 0.2 * x1)
    x2 = jnp.dot(ego * side, wbi_ref[...], preferred_element_type=jnp.float32)
    x2 = x2 + bbi_ref[...]
    bi_emb = jnp.where(x2 >= 0, x2, 0.2 * x2)
    ego2 = sum_emb + bi_emb
    nrm = ego2 / (jnp.sqrt(jnp.sum(ego2 * ego2, axis=1, keepdims=True)) + 1e-12)
    if last:
        out_refs[0][...] = nrm
    else:
        out_refs[0][...] = ego2
        out_refs[1][...] = nrm
        scaled = ego2 * dinv
        out_refs[2][0] = scaled[:, :HD]
        out_refs[2][1] = scaled[:, HD:]


def _make_dense(last):
    out_specs = [pl.BlockSpec((R, D), lambda g: (g, 0))]
    out_shape = [jax.ShapeDtypeStruct((NN, D), jnp.float32)]
    if not last:
        out_specs = out_specs + [
            pl.BlockSpec((R, D), lambda g: (g, 0)),
            pl.BlockSpec((2, R, HD), lambda g: (0, g, 0)),
        ]
        out_shape = out_shape + [
            jax.ShapeDtypeStruct((NN, D), jnp.float32),
            jax.ShapeDtypeStruct((2, NN, HD), jnp.float32),
        ]
    return pl.pallas_call(
        functools.partial(_dense_body, last),
        grid=(G,),
        in_specs=[
            pl.BlockSpec((2, R, HD), lambda g: (0, g, 0)),
            pl.BlockSpec((R, 1), lambda g: (g, 0)),
            pl.BlockSpec((R, D), lambda g: (g, 0)),
            pl.BlockSpec((D, D), lambda g: (0, 0)),
            pl.BlockSpec((1, D), lambda g: (0, 0)),
            pl.BlockSpec((D, D), lambda g: (0, 0)),
            pl.BlockSpec((1, D), lambda g: (0, 0)),
        ],
        out_specs=out_specs,
        out_shape=out_shape,
    )


_dense0 = _make_dense(last=False)
_dense1 = _make_dense(last=True)


def _dot_body(g_ref, y_ref):
    yui = jnp.zeros((B,), jnp.float32)
    yuj = jnp.zeros((B,), jnp.float32)
    for t in range(3):
        u_rows = g_ref[t, 0]
        yui = yui + jnp.sum(u_rows * g_ref[t, 1], axis=-1)
        yuj = yuj + jnp.sum(u_rows * g_ref[t, 2], axis=-1)
    y_ref[...] = jnp.stack([yui, yuj], axis=1)


_tc_dot = pl.pallas_call(
    _dot_body,
    grid=(1,),
    in_specs=[pl.BlockSpec((3, 3, B, D), lambda g: (0, 0, 0, 0))],
    out_specs=pl.BlockSpec((B, 2), lambda g: (0, 0)),
    out_shape=jax.ShapeDtypeStruct((B, 2), jnp.float32),
)


def kernel(emb, W_gc0, b_gc0, W_bi0, b_bi0, W_gc1, b_gc1, W_bi1, b_bi1,
           u, i, j, edge_index):
    ei = edge_index.astype(jnp.int32)
    src = ei[0]
    dst = ei[1]

    dst_pad = jnp.concatenate(
        [dst, jnp.full((PAD,), NN, jnp.int32)]).reshape(NW, EPW)
    degp = _deg_kernel(dst_pad)

    dinv, s_stack = _tc0(degp, emb)

    src2 = jnp.stack([src, src + NN]).reshape(NC, CPR, K)
    dst2 = dst.reshape(CPR, K)

    acc0 = _propagate_kernel(src2, dst2, s_stack.reshape(NC * NN, HD))
    ego1, n0, s_stack1 = _dense0(
        acc0, dinv, emb, W_gc0, b_gc0.reshape(1, D), W_bi0, b_bi0.reshape(1, D))

    acc1 = _propagate_kernel(src2, dst2, s_stack1.reshape(NC * NN, HD))
    (n1,) = _dense1(
        acc1, dinv, ego1, W_gc1, b_gc1.reshape(1, D), W_bi1, b_bi1.reshape(1, D))

    uij = jnp.stack([
        u.astype(jnp.int32),
        NU + i.astype(jnp.int32),
        NU + j.astype(jnp.int32),
    ]).reshape(3, NW, L)
    g = _gather_kernel(emb, n0, n1, uij)
    return _tc_dot(g)


# speedup vs baseline: 31.8084x; 1.3279x over previous
import functools

import jax
import jax.numpy as jnp
from jax import lax
from jax.experimental import pallas as pl
from jax.experimental.pallas import tpu as pltpu
from jax.experimental.pallas import tpu_sc as plsc

NU = 25000
NN = 50000
NE = 800000
D = 64
HD = 32
B = 512
NC = 2
NS = 16
NW = NC * NS
L = 16

EPE = NE // NW
DGRP = EPE // L

K = 250
CPR = NE // K
EPT = NE // NS
NCHUNK = EPT // K
CG = 10
RPT = 3200
ZR = 128

R = 2048
G = 25
NNP = R * G
RF = R * HD // 128
GP = G * RF

_sc_mesh = plsc.VectorSubcoreMesh(
    core_axis_name="c", subcore_axis_name="s", num_cores=NC, num_subcores=NS
)
_sc_params = pltpu.CompilerParams(
    needs_layout_passes=False, use_tc_tiling_on_sc=False)


@functools.partial(
    pl.kernel,
    out_type=jax.ShapeDtypeStruct((NW, NNP), jnp.float32),
    mesh=_sc_mesh,
    compiler_params=_sc_params,
    scratch_types=[
        pltpu.VMEM((EPE + 8,), jnp.int32),
        pltpu.VMEM((NNP,), jnp.float32),
    ],
)
def _deg_kernel(ei_hbm, out_hbm, dst_v, deg_v):
    c = lax.axis_index("c")
    s = lax.axis_index("s")
    wid = s * NC + c

    zeros = jnp.zeros((L,), jnp.float32)

    def zero_it(g, carry):
        deg_v[pl.ds(g * L, L)] = zeros
        return carry

    lax.fori_loop(0, NNP // L, zero_it, 0)

    pltpu.sync_copy(ei_hbm.at[1, pl.ds(wid * EPE, EPE)],
                    dst_v.at[pl.ds(0, EPE)])

    ones = jnp.ones((L,), jnp.float32)

    def acc_it(g, carry):
        idx = dst_v[pl.ds(g * L, L)]
        plsc.addupdate_scatter(deg_v, [idx], ones)
        return carry

    lax.fori_loop(0, DGRP, acc_it, 0)
    idx = dst_v[pl.ds(DGRP * L, L)]
    mask = lax.iota(jnp.int32, L) < (EPE - DGRP * L)
    plsc.addupdate_scatter(deg_v, [idx], ones, mask=mask)

    pltpu.sync_copy(deg_v, out_hbm.at[wid])


@functools.partial(
    pl.kernel,
    out_type=jax.ShapeDtypeStruct((NC, NNP, HD), jnp.float32),
    mesh=_sc_mesh,
    compiler_params=_sc_params,
    scratch_types=[
        pltpu.VMEM((2, CG, K), jnp.int32),
        pltpu.VMEM((2, CG, K), jnp.int32),
        pltpu.VMEM((K, HD), jnp.float32),
        pltpu.VMEM((K, HD), jnp.float32),
        pltpu.VMEM_SHARED((NNP, HD), jnp.float32),
        pltpu.SemaphoreType.DMA,
        pltpu.SemaphoreType.DMA,
    ],
)
def _propagate_kernel(src2_hbm, dst2_hbm, table_hbm, out_hbm,
                      src_v, dst_v, rows0_v, rows1_v, acc_s, sem0, sem1):
    c = lax.axis_index("c")
    s = lax.axis_index("s")
    rows = (rows0_v, rows1_v)
    sems = (sem0, sem1)
    tbl = table_hbm.at[pl.ds(c * NNP, NNP)]

    zeros = jnp.zeros((L,), jnp.float32)

    def zero_buf(i, carry):
        rows0_v[i, pl.ds(0, L)] = zeros
        rows0_v[i, pl.ds(L, L)] = zeros
        return carry

    lax.fori_loop(0, ZR, zero_buf, 0)

    def zero_stripe(r, carry):
        pltpu.sync_copy(rows0_v.at[pl.ds(0, ZR)],
                        acc_s.at[pl.ds(s * RPT + r * ZR, ZR)])
        return carry

    lax.fori_loop(0, RPT // ZR, zero_stripe, 0)
    plsc.subcore_barrier()

    base = s * NCHUNK

    def load_group(o):
        g0 = base + o * CG
        pltpu.sync_copy(src2_hbm.at[pl.ds(g0, CG)], src_v.at[o % 2])
        pltpu.sync_copy(dst2_hbm.at[pl.ds(g0, CG)], dst_v.at[o % 2])

    def issue_gather(m, b):
        o = m // CG
        pltpu.async_copy(tbl.at[src_v.at[o % 2, m % CG]], rows[b], sems[b])

    load_group(0)
    issue_gather(0, 0)
    issue_gather(1, 1)

    def chunk_pair(p, carry):
        def one(m, b):
            o = m // CG
            pltpu.make_async_copy(
                tbl.at[src_v.at[o % 2, m % CG]], rows[b], sems[b]).wait()
            pltpu.sync_copy(rows[b], acc_s.at[dst_v.at[o % 2, m % CG]],
                            add=True)
            m2 = m + 2

            @pl.when(m2 < NCHUNK)
            def _():
                @pl.when(m2 % CG == 0)
                def _():
                    load_group(m2 // CG)
                issue_gather(m2, b)

        one(2 * p, 0)
        one(2 * p + 1, 1)
        return carry

    lax.fori_loop(0, NCHUNK // 2, chunk_pair, 0)
    plsc.subcore_barrier()

    def writeout(r, carry):
        r0 = s * RPT + r * ZR
        pltpu.sync_copy(acc_s.at[pl.ds(r0, ZR)], rows0_v.at[pl.ds(0, ZR)])
        pltpu.sync_copy(rows0_v.at[pl.ds(0, ZR)], out_hbm.at[c, pl.ds(r0, ZR)])
        return carry

    lax.fori_loop(0, RPT // ZR, writeout, 0)


@functools.partial(
    pl.kernel,
    out_type=jax.ShapeDtypeStruct((2, 3, B, 128), jnp.float32),
    mesh=_sc_mesh,
    compiler_params=_sc_params,
    scratch_types=[
        pltpu.VMEM((L,), jnp.int32),
        pltpu.VMEM((L, 128), jnp.float32),
        pltpu.SemaphoreType.DMA,
    ],
)
def _gather_kernel(t0_hbm, t1_hbm, uij_hbm, out_hbm, idx_v, rows_v, sem):
    c = lax.axis_index("c")
    s = lax.axis_index("s")
    wid = s * NC + c
    for q in range(3):
        pltpu.sync_copy(uij_hbm.at[q, wid], idx_v)
        for t, tbl in enumerate((t0_hbm, t1_hbm)):
            pltpu.async_copy(tbl.at[idx_v], rows_v, sem).wait()
            pltpu.sync_copy(rows_v, out_hbm.at[t, q, pl.ds(wid * L, L)])


def _fold(x):
    return jnp.concatenate([x[a * RF:(a + 1) * RF] for a in range(4)], axis=1)


def _unfold(x):
    return jnp.concatenate(
        [x[:, a * HD:(a + 1) * HD] for a in range(4)], axis=0)


def _tc0_body(degp_ref, emb_ref, dinv_ref, s128_ref):
    deg = jnp.sum(degp_ref[...], axis=0)
    dinv = jnp.where(deg > 0, lax.rsqrt(deg), 0.0)
    dinv_ref[...] = dinv
    scaled = emb_ref[...] * dinv[:, None]
    s128_ref[0] = _fold(scaled[:, :HD])
    s128_ref[1] = _fold(scaled[:, HD:])


_tc0 = pl.pallas_call(
    _tc0_body,
    grid=(G,),
    in_specs=[
        pl.BlockSpec((NW, R), lambda g: (0, g)),
        pl.BlockSpec((R, D), lambda g: (g, 0)),
    ],
    out_specs=[
        pl.BlockSpec((R,), lambda g: (g,)),
        pl.BlockSpec((2, RF, 128), lambda g: (0, g, 0)),
    ],
    out_shape=[
        jax.ShapeDtypeStruct((NN,), jnp.float32),
        jax.ShapeDtypeStruct((2, GP, 128), jnp.float32),
    ],
)


def _dense_body(last, acc_ref, dinv_ref, ego_ref, wgc_ref, bgc_ref,
                wbi_ref, bbi_ref, *out_refs):
    dinv = dinv_ref[...][:, None]
    agg = jnp.concatenate(
        [_unfold(acc_ref[0]), _unfold(acc_ref[1])], axis=1) * dinv
    ego = ego_ref[...]
    side = agg + ego
    x1 = jnp.dot(side, wgc_ref[...], preferred_element_type=jnp.float32)
    x1 = x1 + bgc_ref[...]
    sum_emb = jnp.where(x1 >= 0, x1, 0.2 * x1)
    x2 = jnp.dot(ego * side, wbi_ref[...], preferred_element_type=jnp.float32)
    x2 = x2 + bbi_ref[...]
    bi_emb = jnp.where(x2 >= 0, x2, 0.2 * x2)
    ego2 = sum_emb + bi_emb
    nrm = ego2 / (jnp.sqrt(jnp.sum(ego2 * ego2, axis=1, keepdims=True)) + 1e-12)
    if last:
        out_refs[0][...] = jnp.concatenate(
            [nrm, jnp.zeros((R, D), jnp.float32)], axis=1)
    else:
        out_refs[0][...] = ego2
        out_refs[1][...] = jnp.concatenate([ego, nrm], axis=1)
        scaled = ego2 * dinv
        out_refs[2][0] = _fold(scaled[:, :HD])
        out_refs[2][1] = _fold(scaled[:, HD:])


def _make_dense(last):
    if last:
        out_specs = [pl.BlockSpec((R, 128), lambda g: (g, 0))]
        out_shape = [jax.ShapeDtypeStruct((NN, 128), jnp.float32)]
    else:
        out_specs = [
            pl.BlockSpec((R, D), lambda g: (g, 0)),
            pl.BlockSpec((R, 128), lambda g: (g, 0)),
            pl.BlockSpec((2, RF, 128), lambda g: (0, g, 0)),
        ]
        out_shape = [
            jax.ShapeDtypeStruct((NN, D), jnp.float32),
            jax.ShapeDtypeStruct((NN, 128), jnp.float32),
            jax.ShapeDtypeStruct((2, GP, 128), jnp.float32),
        ]
    return pl.pallas_call(
        functools.partial(_dense_body, last),
        grid=(G,),
        in_specs=[
            pl.BlockSpec((2, RF, 128), lambda g: (0, g, 0)),
            pl.BlockSpec((R,), lambda g: (g,)),
            pl.BlockSpec((R, D), lambda g: (g, 0)),
            pl.BlockSpec((D, D), lambda g: (0, 0)),
            pl.BlockSpec((1, D), lambda g: (0, 0)),
            pl.BlockSpec((D, D), lambda g: (0, 0)),
            pl.BlockSpec((1, D), lambda g: (0, 0)),
        ],
        out_specs=out_specs,
        out_shape=out_shape,
    )


_dense0 = _make_dense(last=False)
_dense1 = _make_dense(last=True)


def _dot_body(g_ref, y_ref):
    yui = jnp.zeros((B,), jnp.float32)
    yuj = jnp.zeros((B,), jnp.float32)
    for t in range(2):
        u_rows = g_ref[t, 0]
        yui = yui + jnp.sum(u_rows * g_ref[t, 1], axis=-1)
        yuj = yuj + jnp.sum(u_rows * g_ref[t, 2], axis=-1)
    y_ref[...] = jnp.stack([yui, yuj], axis=1)


_tc_dot = pl.pallas_call(
    _dot_body,
    grid=(1,),
    in_specs=[pl.BlockSpec((2, 3, B, 128), lambda g: (0, 0, 0, 0))],
    out_specs=pl.BlockSpec((B, 2), lambda g: (0, 0)),
    out_shape=jax.ShapeDtypeStruct((B, 2), jnp.float32),
)


def kernel(emb, W_gc0, b_gc0, W_bi0, b_bi0, W_gc1, b_gc1, W_bi1, b_bi1,
           u, i, j, edge_index):
    ei = edge_index.astype(jnp.int32)

    def perm(n):
        g = n >> 11
        r = n & 2047
        return (g << 11) | ((r & 511) << 2) | (r >> 9)

    src2 = perm(ei[0]).reshape(CPR, K)
    dst2 = perm(ei[1]).reshape(CPR, K)

    degp = _deg_kernel(ei)
    dinv, s128 = _tc0(degp, emb)

    acc0 = _propagate_kernel(src2, dst2, s128.reshape(NC * NNP, HD))
    ego1, cat0, s128_1 = _dense0(
        acc0.reshape(NC, GP, 128), dinv, emb,
        W_gc0, b_gc0.reshape(1, D), W_bi0, b_bi0.reshape(1, D))

    acc1 = _propagate_kernel(src2, dst2, s128_1.reshape(NC * NNP, HD))
    (cat1,) = _dense1(
        acc1.reshape(NC, GP, 128), dinv, ego1,
        W_gc1, b_gc1.reshape(1, D), W_bi1, b_bi1.reshape(1, D))

    uij = jnp.stack([
        u.astype(jnp.int32),
        NU + i.astype(jnp.int32),
        NU + j.astype(jnp.int32),
    ]).reshape(3, NW, L)
    g = _gather_kernel(cat0, cat1, uij)
    return _tc_dot(g)


# speedup vs baseline: 39.1894x; 1.2320x over previous
import functools

import jax
import jax.numpy as jnp
from jax import lax
from jax.experimental import pallas as pl
from jax.experimental.pallas import tpu as pltpu
from jax.experimental.pallas import tpu_sc as plsc

NU = 25000
NN = 50000
NE = 800000
D = 64
HD = 32
B = 512
NC = 2
NS = 16
NW = NC * NS
L = 16

EPE = NE // NW
DGRP = EPE // L

K = 200
CPR = NE // K
EPT = NE // NS
NCHUNK = EPT // K
RB = 3
CG = 10
NGRP = NCHUNK // CG
RPT = 3200
ZR = 128

R = 2048
G = 25
NNP = R * G
RF = R * HD // 128
GP = G * RF

_sc_mesh = plsc.VectorSubcoreMesh(
    core_axis_name="c", subcore_axis_name="s", num_cores=NC, num_subcores=NS
)
_sc_params = pltpu.CompilerParams(
    needs_layout_passes=False, use_tc_tiling_on_sc=False)


@functools.partial(
    pl.kernel,
    out_type=jax.ShapeDtypeStruct((NW, NNP), jnp.float32),
    mesh=_sc_mesh,
    compiler_params=_sc_params,
    scratch_types=[
        pltpu.VMEM((EPE + 8,), jnp.int32),
        pltpu.VMEM((NNP,), jnp.float32),
    ],
)
def _deg_kernel(ei_hbm, out_hbm, dst_v, deg_v):
    c = lax.axis_index("c")
    s = lax.axis_index("s")
    wid = s * NC + c

    zeros = jnp.zeros((L,), jnp.float32)

    def zero_it(g, carry):
        deg_v[pl.ds(g * L, L)] = zeros
        return carry

    lax.fori_loop(0, NNP // L, zero_it, 0)

    pltpu.sync_copy(ei_hbm.at[1, pl.ds(wid * EPE, EPE)],
                    dst_v.at[pl.ds(0, EPE)])

    ones = jnp.ones((L,), jnp.float32)

    def acc_it(g, carry):
        idx = dst_v[pl.ds(g * L, L)]
        plsc.addupdate_scatter(deg_v, [idx], ones)
        return carry

    lax.fori_loop(0, DGRP, acc_it, 0)
    idx = dst_v[pl.ds(DGRP * L, L)]
    mask = lax.iota(jnp.int32, L) < (EPE - DGRP * L)
    plsc.addupdate_scatter(deg_v, [idx], ones, mask=mask)

    pltpu.sync_copy(deg_v, out_hbm.at[wid])


@functools.partial(
    pl.kernel,
    out_type=jax.ShapeDtypeStruct((NC, NNP, HD), jnp.float32),
    mesh=_sc_mesh,
    compiler_params=_sc_params,
    scratch_types=[
        pltpu.VMEM((2, CG, K), jnp.int32),
        pltpu.VMEM((2, CG, K), jnp.int32),
        pltpu.VMEM((K, HD), jnp.float32),
        pltpu.VMEM((K, HD), jnp.float32),
        pltpu.VMEM((K, HD), jnp.float32),
        pltpu.VMEM_SHARED((NNP, HD), jnp.float32),
        pltpu.SemaphoreType.DMA,
        pltpu.SemaphoreType.DMA,
        pltpu.SemaphoreType.DMA,
        pltpu.SemaphoreType.DMA,
    ],
)
def _propagate_kernel(src2_hbm, dst2_hbm, table_hbm, out_hbm,
                      src_v, dst_v, rows0_v, rows1_v, rows2_v, acc_s,
                      sem0, sem1, sem2, isem):
    c = lax.axis_index("c")
    s = lax.axis_index("s")
    rows = (rows0_v, rows1_v, rows2_v)
    sems = (sem0, sem1, sem2)
    tbl = table_hbm.at[pl.ds(c * NNP, NNP)]

    zeros = jnp.zeros((L,), jnp.float32)

    def zero_buf(i, carry):
        rows0_v[i, pl.ds(0, L)] = zeros
        rows0_v[i, pl.ds(L, L)] = zeros
        return carry

    lax.fori_loop(0, ZR, zero_buf, 0)

    def zero_stripe(r, carry):
        pltpu.sync_copy(rows0_v.at[pl.ds(0, ZR)],
                        acc_s.at[pl.ds(s * RPT + r * ZR, ZR)])
        return carry

    lax.fori_loop(0, RPT // ZR, zero_stripe, 0)

    base = s * NCHUNK

    def load_group0():
        pltpu.sync_copy(src2_hbm.at[pl.ds(base, CG)], src_v.at[0])
        pltpu.sync_copy(dst2_hbm.at[pl.ds(base, CG)], dst_v.at[0])

    def prefetch_group(o):
        g0 = base + o * CG
        pltpu.async_copy(src2_hbm.at[pl.ds(g0, CG)], src_v.at[o % 2], isem)
        pltpu.async_copy(dst2_hbm.at[pl.ds(g0, CG)], dst_v.at[o % 2], isem)

    def wait_group(o):
        pltpu.make_async_copy(
            src2_hbm.at[pl.ds(base, CG)], src_v.at[o % 2], isem).wait()
        pltpu.make_async_copy(
            dst2_hbm.at[pl.ds(base, CG)], dst_v.at[o % 2], isem).wait()

    def issue_gather(m, b):
        o = m // CG
        pltpu.async_copy(tbl.at[src_v.at[o % 2, m % CG]], rows[b], sems[b])

    load_group0()
    for m in range(RB):
        issue_gather(m, m)
    plsc.subcore_barrier()

    def one(m, b):
        o = m // CG
        pltpu.make_async_copy(
            tbl.at[src_v.at[o % 2, m % CG]], rows[b], sems[b]).wait()
        pltpu.sync_copy(rows[b], acc_s.at[dst_v.at[o % 2, m % CG]],
                        add=True)

        @pl.when(jnp.logical_and(m % CG == 0, o + 1 < NGRP))
        def _():
            prefetch_group(o + 1)

        m3 = m + RB

        @pl.when(m3 < NCHUNK)
        def _():
            @pl.when(m3 % CG == 0)
            def _():
                wait_group(m3 // CG)
            issue_gather(m3, b)

    def ring_step(p, carry):
        one(RB * p, 0)
        one(RB * p + 1, 1)
        one(RB * p + 2, 2)
        return carry

    NFULL = (NCHUNK // RB) * RB
    lax.fori_loop(0, NCHUNK // RB, ring_step, 0)
    for m in range(NFULL, NCHUNK):
        one(m, m % RB)
    plsc.subcore_barrier()

    def writeout(r, carry):
        r0 = s * RPT + r * ZR
        pltpu.sync_copy(acc_s.at[pl.ds(r0, ZR)], rows0_v.at[pl.ds(0, ZR)])
        pltpu.sync_copy(rows0_v.at[pl.ds(0, ZR)], out_hbm.at[c, pl.ds(r0, ZR)])
        return carry

    lax.fori_loop(0, RPT // ZR, writeout, 0)


@functools.partial(
    pl.kernel,
    out_type=jax.ShapeDtypeStruct((2, 3, B, 128), jnp.float32),
    mesh=_sc_mesh,
    compiler_params=_sc_params,
    scratch_types=[
        pltpu.VMEM((L,), jnp.int32),
        pltpu.VMEM((L, 128), jnp.float32),
        pltpu.SemaphoreType.DMA,
    ],
)
def _gather_kernel(t0_hbm, t1_hbm, uij_hbm, out_hbm, idx_v, rows_v, sem):
    c = lax.axis_index("c")
    s = lax.axis_index("s")
    wid = s * NC + c
    for q in range(3):
        pltpu.sync_copy(uij_hbm.at[q, wid], idx_v)
        for t, tbl in enumerate((t0_hbm, t1_hbm)):
            pltpu.async_copy(tbl.at[idx_v], rows_v, sem).wait()
            pltpu.sync_copy(rows_v, out_hbm.at[t, q, pl.ds(wid * L, L)])


def _fold(x):
    return jnp.concatenate([x[a * RF:(a + 1) * RF] for a in range(4)], axis=1)


def _unfold(x):
    return jnp.concatenate(
        [x[:, a * HD:(a + 1) * HD] for a in range(4)], axis=0)


def _tc0_body(degp_ref, emb_ref, dinv_ref, s128_ref):
    deg = jnp.sum(degp_ref[...], axis=0)
    dinv = jnp.where(deg > 0, lax.rsqrt(deg), 0.0)
    dinv_ref[...] = dinv
    scaled = emb_ref[...] * dinv[:, None]
    s128_ref[0] = _fold(scaled[:, :HD])
    s128_ref[1] = _fold(scaled[:, HD:])


_tc0 = pl.pallas_call(
    _tc0_body,
    grid=(G,),
    in_specs=[
        pl.BlockSpec((NW, R), lambda g: (0, g)),
        pl.BlockSpec((R, D), lambda g: (g, 0)),
    ],
    out_specs=[
        pl.BlockSpec((R,), lambda g: (g,)),
        pl.BlockSpec((2, RF, 128), lambda g: (0, g, 0)),
    ],
    out_shape=[
        jax.ShapeDtypeStruct((NN,), jnp.float32),
        jax.ShapeDtypeStruct((2, GP, 128), jnp.float32),
    ],
)


def _dense_body(last, acc_ref, dinv_ref, ego_ref, wgc_ref, bgc_ref,
                wbi_ref, bbi_ref, *out_refs):
    dinv = dinv_ref[...][:, None]
    agg = jnp.concatenate(
        [_unfold(acc_ref[0]), _unfold(acc_ref[1])], axis=1) * dinv
    ego = ego_ref[...]
    side = agg + ego
    x1 = jnp.dot(side, wgc_ref[...], preferred_element_type=jnp.float32)
    x1 = x1 + bgc_ref[...]
    sum_emb = jnp.where(x1 >= 0, x1, 0.2 * x1)
    x2 = jnp.dot(ego * side, wbi_ref[...], preferred_element_type=jnp.float32)
    x2 = x2 + bbi_ref[...]
    bi_emb = jnp.where(x2 >= 0, x2, 0.2 * x2)
    ego2 = sum_emb + bi_emb
    nrm = ego2 / (jnp.sqrt(jnp.sum(ego2 * ego2, axis=1, keepdims=True)) + 1e-12)
    if last:
        out_refs[0][...] = jnp.concatenate(
            [nrm, jnp.zeros((R, D), jnp.float32)], axis=1)
    else:
        out_refs[0][...] = ego2
        out_refs[1][...] = jnp.concatenate([ego, nrm], axis=1)
        scaled = ego2 * dinv
        out_refs[2][0] = _fold(scaled[:, :HD])
        out_refs[2][1] = _fold(scaled[:, HD:])


def _make_dense(last):
    if last:
        out_specs = [pl.BlockSpec((R, 128), lambda g: (g, 0))]
        out_shape = [jax.ShapeDtypeStruct((NN, 128), jnp.float32)]
    else:
        out_specs = [
            pl.BlockSpec((R, D), lambda g: (g, 0)),
            pl.BlockSpec((R, 128), lambda g: (g, 0)),
            pl.BlockSpec((2, RF, 128), lambda g: (0, g, 0)),
        ]
        out_shape = [
            jax.ShapeDtypeStruct((NN, D), jnp.float32),
            jax.ShapeDtypeStruct((NN, 128), jnp.float32),
            jax.ShapeDtypeStruct((2, GP, 128), jnp.float32),
        ]
    return pl.pallas_call(
        functools.partial(_dense_body, last),
        grid=(G,),
        in_specs=[
            pl.BlockSpec((2, RF, 128), lambda g: (0, g, 0)),
            pl.BlockSpec((R,), lambda g: (g,)),
            pl.BlockSpec((R, D), lambda g: (g, 0)),
            pl.BlockSpec((D, D), lambda g: (0, 0)),
            pl.BlockSpec((1, D), lambda g: (0, 0)),
            pl.BlockSpec((D, D), lambda g: (0, 0)),
            pl.BlockSpec((1, D), lambda g: (0, 0)),
        ],
        out_specs=out_specs,
        out_shape=out_shape,
    )


_dense0 = _make_dense(last=False)
_dense1 = _make_dense(last=True)


def _dot_body(g_ref, y_ref):
    yui = jnp.zeros((B,), jnp.float32)
    yuj = jnp.zeros((B,), jnp.float32)
    for t in range(2):
        u_rows = g_ref[t, 0]
        yui = yui + jnp.sum(u_rows * g_ref[t, 1], axis=-1)
        yuj = yuj + jnp.sum(u_rows * g_ref[t, 2], axis=-1)
    y_ref[...] = jnp.stack([yui, yuj], axis=1)


_tc_dot = pl.pallas_call(
    _dot_body,
    grid=(1,),
    in_specs=[pl.BlockSpec((2, 3, B, 128), lambda g: (0, 0, 0, 0))],
    out_specs=pl.BlockSpec((B, 2), lambda g: (0, 0)),
    out_shape=jax.ShapeDtypeStruct((B, 2), jnp.float32),
)


def kernel(emb, W_gc0, b_gc0, W_bi0, b_bi0, W_gc1, b_gc1, W_bi1, b_bi1,
           u, i, j, edge_index):
    ei = edge_index.astype(jnp.int32)

    def perm(n):
        g = n >> 11
        r = n & 2047
        return (g << 11) | ((r & 511) << 2) | (r >> 9)

    src2 = perm(ei[0]).reshape(CPR, K)
    dst2 = perm(ei[1]).reshape(CPR, K)

    degp = _deg_kernel(ei)
    dinv, s128 = _tc0(degp, emb)

    acc0 = _propagate_kernel(src2, dst2, s128.reshape(NC * NNP, HD))
    ego1, cat0, s128_1 = _dense0(
        acc0.reshape(NC, GP, 128), dinv, emb,
        W_gc0, b_gc0.reshape(1, D), W_bi0, b_bi0.reshape(1, D))

    acc1 = _propagate_kernel(src2, dst2, s128_1.reshape(NC * NNP, HD))
    (cat1,) = _dense1(
        acc1.reshape(NC, GP, 128), dinv, ego1,
        W_gc1, b_gc1.reshape(1, D), W_bi1, b_bi1.reshape(1, D))

    uij = jnp.stack([
        u.astype(jnp.int32),
        NU + i.astype(jnp.int32),
        NU + j.astype(jnp.int32),
    ]).reshape(3, NW, L)
    g = _gather_kernel(cat0, cat1, uij)
    return _tc_dot(g)


# speedup vs baseline: 41.6164x; 1.0619x over previous
import functools

import jax
import jax.numpy as jnp
from jax import lax
from jax.experimental import pallas as pl
from jax.experimental.pallas import tpu as pltpu
from jax.experimental.pallas import tpu_sc as plsc

NU = 25000
NN = 50000
NE = 800000
D = 64
HD = 32
B = 512
NC = 2
NS = 16
NW = NC * NS
L = 16

EPE = NE // NW
DGRP = EPE // L

K = 200
CPR = NE // K
EPT = NE // NS
NCHUNK = EPT // K
RB = 3
CG = 10
NGRP = NCHUNK // CG
RPT = 3200
ZR = 128

R = 2048
G = 25
NNP = R * G
RF = R * HD // 128
GP = G * RF

_sc_mesh = plsc.VectorSubcoreMesh(
    core_axis_name="c", subcore_axis_name="s", num_cores=NC, num_subcores=NS
)
_sc_params = pltpu.CompilerParams(
    needs_layout_passes=False, use_tc_tiling_on_sc=False)


@functools.partial(
    pl.kernel,
    out_type=jax.ShapeDtypeStruct((NW, NNP), jnp.float32),
    mesh=_sc_mesh,
    compiler_params=_sc_params,
    scratch_types=[
        pltpu.VMEM((EPE + 8,), jnp.int32),
        pltpu.VMEM((NNP,), jnp.float32),
    ],
)
def _deg_kernel(dst_hbm, out_hbm, dst_v, deg_v):
    c = lax.axis_index("c")
    s = lax.axis_index("s")
    wid = s * NC + c

    zeros = jnp.zeros((L,), jnp.float32)

    def zero_it(g, carry):
        deg_v[pl.ds(g * L, L)] = zeros
        return carry

    lax.fori_loop(0, NNP // L, zero_it, 0)

    pltpu.sync_copy(dst_hbm.at[pl.ds(wid * EPE, EPE)],
                    dst_v.at[pl.ds(0, EPE)])

    ones = jnp.ones((L,), jnp.float32)

    def acc_it(g, carry):
        idx = dst_v[pl.ds(g * L, L)]
        plsc.addupdate_scatter(deg_v, [idx], ones)
        return carry

    lax.fori_loop(0, DGRP, acc_it, 0)
    idx = dst_v[pl.ds(DGRP * L, L)]
    mask = lax.iota(jnp.int32, L) < (EPE - DGRP * L)
    plsc.addupdate_scatter(deg_v, [idx], ones, mask=mask)

    pltpu.sync_copy(deg_v, out_hbm.at[wid])


@functools.partial(
    pl.kernel,
    out_type=jax.ShapeDtypeStruct((NC, NNP, HD), jnp.float32),
    mesh=_sc_mesh,
    compiler_params=_sc_params,
    scratch_types=[
        pltpu.VMEM((2, CG, K), jnp.int32),
        pltpu.VMEM((2, CG, K), jnp.int32),
        pltpu.VMEM((K, HD), jnp.float32),
        pltpu.VMEM((K, HD), jnp.float32),
        pltpu.VMEM((K, HD), jnp.float32),
        pltpu.VMEM_SHARED((NNP, HD), jnp.float32),
        pltpu.SemaphoreType.DMA,
        pltpu.SemaphoreType.DMA,
        pltpu.SemaphoreType.DMA,
        pltpu.SemaphoreType.DMA,
    ],
)
def _propagate_kernel(src2_hbm, dst2_hbm, table_hbm, out_hbm,
                      src_v, dst_v, rows0_v, rows1_v, rows2_v, acc_s,
                      sem0, sem1, sem2, isem):
    c = lax.axis_index("c")
    s = lax.axis_index("s")
    rows = (rows0_v, rows1_v, rows2_v)
    sems = (sem0, sem1, sem2)
    tbl = table_hbm.at[pl.ds(c * NNP, NNP)]

    zeros = jnp.zeros((L,), jnp.float32)

    def zero_buf(i, carry):
        rows0_v[i, pl.ds(0, L)] = zeros
        rows0_v[i, pl.ds(L, L)] = zeros
        return carry

    lax.fori_loop(0, ZR, zero_buf, 0)

    def zero_stripe(r, carry):
        pltpu.sync_copy(rows0_v.at[pl.ds(0, ZR)],
                        acc_s.at[pl.ds(s * RPT + r * ZR, ZR)])
        return carry

    lax.fori_loop(0, RPT // ZR, zero_stripe, 0)

    base = s * NCHUNK

    def load_group0():
        pltpu.sync_copy(src2_hbm.at[pl.ds(base, CG)], src_v.at[0])
        pltpu.sync_copy(dst2_hbm.at[pl.ds(base, CG)], dst_v.at[0])

    def prefetch_group(o):
        g0 = base + o * CG
        pltpu.async_copy(src2_hbm.at[pl.ds(g0, CG)], src_v.at[o % 2], isem)
        pltpu.async_copy(dst2_hbm.at[pl.ds(g0, CG)], dst_v.at[o % 2], isem)

    def wait_group(o):
        pltpu.make_async_copy(
            src2_hbm.at[pl.ds(base, CG)], src_v.at[o % 2], isem).wait()
        pltpu.make_async_copy(
            dst2_hbm.at[pl.ds(base, CG)], dst_v.at[o % 2], isem).wait()

    def issue_gather(m, b):
        o = m // CG
        pltpu.async_copy(tbl.at[src_v.at[o % 2, m % CG]], rows[b], sems[b])

    load_group0()
    for m in range(RB):
        issue_gather(m, m)
    plsc.subcore_barrier()

    def one(m, b):
        o = m // CG
        pltpu.make_async_copy(
            tbl.at[src_v.at[o % 2, m % CG]], rows[b], sems[b]).wait()
        pltpu.sync_copy(rows[b], acc_s.at[dst_v.at[o % 2, m % CG]],
                        add=True)

        @pl.when(jnp.logical_and(m % CG == 0, o + 1 < NGRP))
        def _():
            prefetch_group(o + 1)

        m3 = m + RB

        @pl.when(m3 < NCHUNK)
        def _():
            @pl.when(m3 % CG == 0)
            def _():
                wait_group(m3 // CG)
            issue_gather(m3, b)

    def ring_step(p, carry):
        one(RB * p, 0)
        one(RB * p + 1, 1)
        one(RB * p + 2, 2)
        return carry

    NFULL = (NCHUNK // RB) * RB
    lax.fori_loop(0, NCHUNK // RB, ring_step, 0)
    for m in range(NFULL, NCHUNK):
        one(m, m % RB)
    plsc.subcore_barrier()

    def writeout(r, carry):
        r0 = s * RPT + r * ZR
        pltpu.sync_copy(acc_s.at[pl.ds(r0, ZR)], rows0_v.at[pl.ds(0, ZR)])
        pltpu.sync_copy(rows0_v.at[pl.ds(0, ZR)], out_hbm.at[c, pl.ds(r0, ZR)])
        return carry

    lax.fori_loop(0, RPT // ZR, writeout, 0)


@functools.partial(
    pl.kernel,
    out_type=jax.ShapeDtypeStruct((2, 3, B, 128), jnp.float32),
    mesh=_sc_mesh,
    compiler_params=_sc_params,
    scratch_types=[
        pltpu.VMEM((L,), jnp.int32),
        pltpu.VMEM((L, 128), jnp.float32),
        pltpu.SemaphoreType.DMA,
    ],
)
def _gather_kernel(t0_hbm, t1_hbm, uij_hbm, out_hbm, idx_v, rows_v, sem):
    c = lax.axis_index("c")
    s = lax.axis_index("s")
    wid = s * NC + c
    for q in range(3):
        pltpu.sync_copy(uij_hbm.at[q, wid], idx_v)
        for t, tbl in enumerate((t0_hbm, t1_hbm)):
            pltpu.async_copy(tbl.at[idx_v], rows_v, sem).wait()
            pltpu.sync_copy(rows_v, out_hbm.at[t, q, pl.ds(wid * L, L)])


EB = NE


def _perm(n):
    g = n >> 11
    r = n & 2047
    return (g << 11) | ((r & 511) << 2) | (r >> 9)


def _prep_body(ei_ref, srcp_ref, dstp_ref, dstraw_ref):
    s = ei_ref[0]
    d = ei_ref[1]
    srcp_ref[...] = _perm(s)
    dstp_ref[...] = _perm(d)
    dstraw_ref[...] = d


_prep = pl.pallas_call(
    _prep_body,
    grid=(1,),
    in_specs=[pl.BlockSpec((2, EB), lambda g: (0, 0))],
    out_specs=[
        pl.BlockSpec((EB,), lambda g: (0,)),
        pl.BlockSpec((EB,), lambda g: (0,)),
        pl.BlockSpec((EB,), lambda g: (0,)),
    ],
    out_shape=[
        jax.ShapeDtypeStruct((NE,), jnp.int32),
        jax.ShapeDtypeStruct((NE,), jnp.int32),
        jax.ShapeDtypeStruct((NE,), jnp.int32),
    ],
)


def _fold(x):
    return jnp.concatenate([x[a * RF:(a + 1) * RF] for a in range(4)], axis=1)


def _unfold(x):
    return jnp.concatenate(
        [x[:, a * HD:(a + 1) * HD] for a in range(4)], axis=0)


def _tc0_body(degp_ref, emb_ref, dinv_ref, s128_ref):
    deg = jnp.sum(degp_ref[...], axis=0)
    dinv = jnp.where(deg > 0, lax.rsqrt(deg), 0.0)
    dinv_ref[...] = dinv
    scaled = emb_ref[...] * dinv[:, None]
    s128_ref[0] = _fold(scaled[:, :HD])
    s128_ref[1] = _fold(scaled[:, HD:])


_tc0 = pl.pallas_call(
    _tc0_body,
    grid=(G,),
    in_specs=[
        pl.BlockSpec((NW, R), lambda g: (0, g)),
        pl.BlockSpec((R, D), lambda g: (g, 0)),
    ],
    out_specs=[
        pl.BlockSpec((R,), lambda g: (g,)),
        pl.BlockSpec((2, RF, 128), lambda g: (0, g, 0)),
    ],
    out_shape=[
        jax.ShapeDtypeStruct((NN,), jnp.float32),
        jax.ShapeDtypeStruct((2, GP, 128), jnp.float32),
    ],
)


def _dense_body(last, acc_ref, dinv_ref, ego_ref, wgc_ref, bgc_ref,
                wbi_ref, bbi_ref, *out_refs):
    dinv = dinv_ref[...][:, None]
    agg = jnp.concatenate(
        [_unfold(acc_ref[0]), _unfold(acc_ref[1])], axis=1) * dinv
    ego = ego_ref[...]
    side = agg + ego
    x1 = jnp.dot(side, wgc_ref[...], preferred_element_type=jnp.float32)
    x1 = x1 + bgc_ref[...]
    sum_emb = jnp.where(x1 >= 0, x1, 0.2 * x1)
    x2 = jnp.dot(ego * side, wbi_ref[...], preferred_element_type=jnp.float32)
    x2 = x2 + bbi_ref[...]
    bi_emb = jnp.where(x2 >= 0, x2, 0.2 * x2)
    ego2 = sum_emb + bi_emb
    nrm = ego2 / (jnp.sqrt(jnp.sum(ego2 * ego2, axis=1, keepdims=True)) + 1e-12)
    if last:
        out_refs[0][...] = jnp.concatenate(
            [nrm, jnp.zeros((R, D), jnp.float32)], axis=1)
    else:
        out_refs[0][...] = ego2
        out_refs[1][...] = jnp.concatenate([ego, nrm], axis=1)
        scaled = ego2 * dinv
        out_refs[2][0] = _fold(scaled[:, :HD])
        out_refs[2][1] = _fold(scaled[:, HD:])


def _make_dense(last):
    if last:
        out_specs = [pl.BlockSpec((R, 128), lambda g: (g, 0))]
        out_shape = [jax.ShapeDtypeStruct((NN, 128), jnp.float32)]
    else:
        out_specs = [
            pl.BlockSpec((R, D), lambda g: (g, 0)),
            pl.BlockSpec((R, 128), lambda g: (g, 0)),
            pl.BlockSpec((2, RF, 128), lambda g: (0, g, 0)),
        ]
        out_shape = [
            jax.ShapeDtypeStruct((NN, D), jnp.float32),
            jax.ShapeDtypeStruct((NN, 128), jnp.float32),
            jax.ShapeDtypeStruct((2, GP, 128), jnp.float32),
        ]
    return pl.pallas_call(
        functools.partial(_dense_body, last),
        grid=(G,),
        in_specs=[
            pl.BlockSpec((2, RF, 128), lambda g: (0, g, 0)),
            pl.BlockSpec((R,), lambda g: (g,)),
            pl.BlockSpec((R, D), lambda g: (g, 0)),
            pl.BlockSpec((D, D), lambda g: (0, 0)),
            pl.BlockSpec((1, D), lambda g: (0, 0)),
            pl.BlockSpec((D, D), lambda g: (0, 0)),
            pl.BlockSpec((1, D), lambda g: (0, 0)),
        ],
        out_specs=out_specs,
        out_shape=out_shape,
    )


_dense0 = _make_dense(last=False)
_dense1 = _make_dense(last=True)


def _dot_body(g_ref, y_ref):
    yui = jnp.zeros((B,), jnp.float32)
    yuj = jnp.zeros((B,), jnp.float32)
    for t in range(2):
        u_rows = g_ref[t, 0]
        yui = yui + jnp.sum(u_rows * g_ref[t, 1], axis=-1)
        yuj = yuj + jnp.sum(u_rows * g_ref[t, 2], axis=-1)
    y_ref[...] = jnp.stack([yui, yuj], axis=1)


_tc_dot = pl.pallas_call(
    _dot_body,
    grid=(1,),
    in_specs=[pl.BlockSpec((2, 3, B, 128), lambda g: (0, 0, 0, 0))],
    out_specs=pl.BlockSpec((B, 2), lambda g: (0, 0)),
    out_shape=jax.ShapeDtypeStruct((B, 2), jnp.float32),
)


def kernel(emb, W_gc0, b_gc0, W_bi0, b_bi0, W_gc1, b_gc1, W_bi1, b_bi1,
           u, i, j, edge_index):
    ei = edge_index.astype(jnp.int32)
    srcp, dstp, dstraw = _prep(ei)
    src2 = srcp.reshape(CPR, K)
    dst2 = dstp.reshape(CPR, K)

    degp = _deg_kernel(dstraw)
    dinv, s128 = _tc0(degp, emb)

    acc0 = _propagate_kernel(src2, dst2, s128.reshape(NC * NNP, HD))
    ego1, cat0, s128_1 = _dense0(
        acc0.reshape(NC, GP, 128), dinv, emb,
        W_gc0, b_gc0.reshape(1, D), W_bi0, b_bi0.reshape(1, D))

    acc1 = _propagate_kernel(src2, dst2, s128_1.reshape(NC * NNP, HD))
    (cat1,) = _dense1(
        acc1.reshape(NC, GP, 128), dinv, ego1,
        W_gc1, b_gc1.reshape(1, D), W_bi1, b_bi1.reshape(1, D))

    uij = jnp.stack([
        u.astype(jnp.int32),
        NU + i.astype(jnp.int32),
        NU + j.astype(jnp.int32),
    ]).reshape(3, NW, L)
    g = _gather_kernel(cat0, cat1, uij)
    return _tc_dot(g)
